# baseline (speedup 1.0000x reference)
import functools

import jax
import jax.numpy as jnp
from jax import lax
from jax.experimental import pallas as pl
from jax.experimental.pallas import tpu as pltpu
from jax.experimental.pallas import tpu_sc as plsc

N = 10000
E = 320000
D = 128
BN_EPS = 1e-5

NC = 2
NS = 16
NW = NC * NS
EDGES_PER_W = E // NW
CHUNK = 80
CHUNKS_PER_W = EDGES_PER_W // CHUNK
ROWS_PER_TILE = 624
TAIL_ROWS = N - NS * ROWS_PER_TILE


def _sc_aggregate(x, src_c, dst_c, zeros_init):
    mesh = plsc.VectorSubcoreMesh(core_axis_name="c", subcore_axis_name="s",
                                  num_cores=NC, num_subcores=NS)

    @functools.partial(
        pl.kernel,
        out_type=jax.ShapeDtypeStruct((NC, N, D), jnp.float32),
        mesh=mesh,
        scratch_types=[
            pltpu.VMEM_SHARED((N, D), jnp.float32),
            pltpu.VMEM((EDGES_PER_W,), jnp.int32),
            pltpu.VMEM((CHUNKS_PER_W, CHUNK), jnp.int32),
            pltpu.VMEM((2, CHUNK, D), jnp.float32),
            pltpu.SemaphoreType.DMA,
            pltpu.SemaphoreType.DMA,
        ],
    )
    def agg_kernel(x_hbm, src_hbm, dst_hbm, zero_hbm, out_hbm,
                   acc, src_v, dst_v, rows_v, gsem, ssem):
        c = lax.axis_index("c")
        s = lax.axis_index("s")
        w = c * NS + s

        r0 = s * ROWS_PER_TILE

        @pl.when(c == 0)
        def _():
            pltpu.sync_copy(x_hbm.at[pl.ds(r0, ROWS_PER_TILE)],
                            acc.at[pl.ds(r0, ROWS_PER_TILE)])

            @pl.when(s == 0)
            def _():
                pltpu.sync_copy(x_hbm.at[pl.ds(NS * ROWS_PER_TILE, TAIL_ROWS)],
                                acc.at[pl.ds(NS * ROWS_PER_TILE, TAIL_ROWS)])

        @pl.when(c != 0)
        def _():
            pltpu.sync_copy(zero_hbm.at[pl.ds(r0, ROWS_PER_TILE)],
                            acc.at[pl.ds(r0, ROWS_PER_TILE)])

            @pl.when(s == 0)
            def _():
                pltpu.sync_copy(
                    zero_hbm.at[pl.ds(NS * ROWS_PER_TILE, TAIL_ROWS)],
                    acc.at[pl.ds(NS * ROWS_PER_TILE, TAIL_ROWS)])

        pltpu.sync_copy(src_hbm.at[w], src_v)
        pltpu.sync_copy(dst_hbm.at[w], dst_v)
        plsc.subcore_barrier()

        def src_slice(j):
            return src_v.at[pl.ds(pl.multiple_of(j * CHUNK, 8), CHUNK)]

        pltpu.async_copy(x_hbm.at[src_slice(0)], rows_v.at[0], gsem)

        def body(j, _):
            buf = lax.rem(j, 2)
            nbuf = lax.rem(j + 1, 2)

            pltpu.make_async_copy(x_hbm.at[src_slice(j)], rows_v.at[buf],
                                  gsem).wait()
            pltpu.async_copy(rows_v.at[buf], acc.at[dst_v.at[j]], ssem,
                             add=True)

            @pl.when(j >= 1)
            def _():
                pltpu.make_async_copy(rows_v.at[nbuf],
                                      acc.at[dst_v.at[j - 1]], ssem).wait()

            @pl.when(j + 1 < CHUNKS_PER_W)
            def _():
                pltpu.async_copy(x_hbm.at[src_slice(j + 1)], rows_v.at[nbuf],
                                 gsem)
            return 0

        lax.fori_loop(0, CHUNKS_PER_W, body, 0)
        pltpu.make_async_copy(rows_v.at[lax.rem(CHUNKS_PER_W - 1, 2)],
                              acc.at[dst_v.at[CHUNKS_PER_W - 1]], ssem).wait()
        plsc.subcore_barrier()

        pltpu.sync_copy(acc.at[pl.ds(r0, ROWS_PER_TILE)],
                        out_hbm.at[c, pl.ds(r0, ROWS_PER_TILE)])

        @pl.when(s == 0)
        def _():
            pltpu.sync_copy(acc.at[pl.ds(NS * ROWS_PER_TILE, TAIL_ROWS)],
                            out_hbm.at[c, pl.ds(NS * ROWS_PER_TILE, TAIL_ROWS)])

    return agg_kernel(x, src_c, dst_c, zeros_init)


def _mlp_body(parts_ref, w1_ref, b1_ref, gamma_ref, beta_ref, w2_ref, b2_ref,
              out_ref):
    h = parts_ref[0] + parts_ref[1]
    h1 = jnp.dot(h, w1_ref[...], preferred_element_type=jnp.float32)
    h1 = h1 + b1_ref[...]
    mean = jnp.mean(h1, axis=0, keepdims=True)
    cent = h1 - mean
    var = jnp.mean(cent * cent, axis=0, keepdims=True)
    hn = gamma_ref[...] * cent * lax.rsqrt(var + BN_EPS) + beta_ref[...]
    hr = jnp.maximum(hn, 0.0)
    out = jnp.dot(hr, w2_ref[...], preferred_element_type=jnp.float32)
    out_ref[...] = out + b2_ref[...]


def kernel(x, edge_index, edge_attr, w1, b1, gamma, beta, w2, b2):
    del edge_attr
    src_c = edge_index[0].reshape(NW, EDGES_PER_W)
    dst_c = edge_index[1].reshape(NW, CHUNKS_PER_W, CHUNK)
    zeros_init = jnp.zeros((N, D), dtype=jnp.float32)

    parts = _sc_aggregate(x, src_c, dst_c, zeros_init)

    out = pl.pallas_call(
        _mlp_body,
        out_shape=jax.ShapeDtypeStruct((N, D), jnp.float32),
    )(parts, w1, b1.reshape(1, D), gamma.reshape(1, D), beta.reshape(1, D),
      w2, b2.reshape(1, D))
    return out


# speedup vs baseline: 1.3424x; 1.3424x over previous
import functools

import jax
import jax.numpy as jnp
from jax import lax
from jax.experimental import pallas as pl
from jax.experimental.pallas import tpu as pltpu
from jax.experimental.pallas import tpu_sc as plsc

N = 10000
E = 320000
D = 128
BN_EPS = 1e-5

NC = 2
NS = 16
NW = NC * NS
CHUNK = 128
SHARE = 10240
FULL_CHUNKS = SHARE // CHUNK
TAIL_SHARE = E - (NW - 1) * SHARE
TAIL_CHUNKS = TAIL_SHARE // CHUNK
PHASE = 40
ROWS_PER_TILE = 624
TAIL_ROWS = N - NS * ROWS_PER_TILE


def _sc_aggregate(x, src_flat, dst_flat):
    mesh = plsc.VectorSubcoreMesh(core_axis_name="c", subcore_axis_name="s",
                                  num_cores=NC, num_subcores=NS)

    @functools.partial(
        pl.kernel,
        out_type=jax.ShapeDtypeStruct((NC, N, D), jnp.float32),
        mesh=mesh,
        scratch_types=[
            pltpu.VMEM_SHARED((N, D), jnp.float32),
            pltpu.VMEM((SHARE,), jnp.int32),
            pltpu.VMEM((PHASE * CHUNK,), jnp.int32),
            pltpu.VMEM((CHUNK,), jnp.int32),
            pltpu.VMEM((2, CHUNK, D), jnp.float32),
            pltpu.SemaphoreType.DMA,
        ],
    )
    def agg_kernel(x_hbm, src_hbm, dst_hbm, out_hbm,
                   acc, src_v, dst_v, cbuf, rows_v, gsem):
        c = lax.axis_index("c")
        s = lax.axis_index("s")
        w = c * NS + s
        base = pl.multiple_of(w * SHARE, CHUNK)
        nchunks = jnp.where(w == NW - 1, TAIL_CHUNKS, FULL_CHUNKS)

        r0 = s * ROWS_PER_TILE
        pltpu.sync_copy(x_hbm.at[pl.ds(r0, ROWS_PER_TILE)],
                        acc.at[pl.ds(r0, ROWS_PER_TILE)])

        @pl.when(s == 0)
        def _():
            pltpu.sync_copy(x_hbm.at[pl.ds(NS * ROWS_PER_TILE, TAIL_ROWS)],
                            acc.at[pl.ds(NS * ROWS_PER_TILE, TAIL_ROWS)])

        @pl.when(w < NW - 1)
        def _():
            pltpu.sync_copy(src_hbm.at[pl.ds(base, SHARE)], src_v)
            pltpu.sync_copy(dst_hbm.at[pl.ds(base, PHASE * CHUNK)], dst_v)

        @pl.when(w == NW - 1)
        def _():
            pltpu.sync_copy(src_hbm.at[pl.ds(base, TAIL_SHARE)],
                            src_v.at[pl.ds(0, TAIL_SHARE)])
            pltpu.sync_copy(dst_hbm.at[pl.ds(base, TAIL_SHARE)],
                            dst_v.at[pl.ds(0, TAIL_SHARE)])

        plsc.subcore_barrier()

        def src_slice(j):
            return src_v.at[pl.ds(pl.multiple_of(j * CHUNK, 8), CHUNK)]

        pltpu.async_copy(x_hbm.at[src_slice(0)], rows_v.at[0], gsem)

        def body(j, _):
            @pl.when(j == PHASE)
            def _():
                pltpu.sync_copy(
                    dst_hbm.at[pl.ds(base + PHASE * CHUNK, PHASE * CHUNK)],
                    dst_v)

            @pl.when(j + 1 < nchunks)
            def _():
                pltpu.async_copy(x_hbm.at[src_slice(j + 1)],
                                 rows_v.at[lax.rem(j + 1, 2)], gsem)

            jp = lax.rem(j, PHASE)
            for k in range(CHUNK // 16):
                cbuf[pl.ds(k * 16, 16)] = dst_v[pl.ds(jp * CHUNK + k * 16, 16)]

            pltpu.make_async_copy(x_hbm.at[src_slice(j)],
                                  rows_v.at[lax.rem(j, 2)], gsem).wait()
            pltpu.sync_copy(rows_v.at[lax.rem(j, 2)], acc.at[cbuf], add=True)
            return 0

        lax.fori_loop(0, nchunks, body, 0)
        plsc.subcore_barrier()

        pltpu.sync_copy(acc.at[pl.ds(r0, ROWS_PER_TILE)],
                        out_hbm.at[c, pl.ds(r0, ROWS_PER_TILE)])

        @pl.when(s == 0)
        def _():
            pltpu.sync_copy(acc.at[pl.ds(NS * ROWS_PER_TILE, TAIL_ROWS)],
                            out_hbm.at[c, pl.ds(NS * ROWS_PER_TILE, TAIL_ROWS)])

    return agg_kernel(x, src_flat, dst_flat)


def _mlp_body(parts_ref, x_ref, w1_ref, b1_ref, gamma_ref, beta_ref, w2_ref,
              b2_ref, out_ref):
    h = parts_ref[0] + parts_ref[1] - x_ref[...]
    h1 = jnp.dot(h, w1_ref[...], preferred_element_type=jnp.float32)
    h1 = h1 + b1_ref[...]
    mean = jnp.mean(h1, axis=0, keepdims=True)
    cent = h1 - mean
    var = jnp.mean(cent * cent, axis=0, keepdims=True)
    hn = gamma_ref[...] * cent * lax.rsqrt(var + BN_EPS) + beta_ref[...]
    hr = jnp.maximum(hn, 0.0)
    out = jnp.dot(hr, w2_ref[...], preferred_element_type=jnp.float32)
    out_ref[...] = out + b2_ref[...]


def kernel(x, edge_index, edge_attr, w1, b1, gamma, beta, w2, b2):
    del edge_attr
    src_flat = edge_index[0]
    dst_flat = edge_index[1]

    parts = _sc_aggregate(x, src_flat, dst_flat)

    out = pl.pallas_call(
        _mlp_body,
        out_shape=jax.ShapeDtypeStruct((N, D), jnp.float32),
    )(parts, x, w1, b1.reshape(1, D), gamma.reshape(1, D), beta.reshape(1, D),
      w2, b2.reshape(1, D))
    return out


# speedup vs baseline: 1.4647x; 1.0912x over previous
import functools

import jax
import jax.numpy as jnp
from jax import lax
from jax.experimental import pallas as pl
from jax.experimental.pallas import tpu as pltpu
from jax.experimental.pallas import tpu_sc as plsc

N = 10000
E = 320000
D = 128
BN_EPS = 1e-5

NC = 2
NS = 16
NW = NC * NS
CHUNK = 128
SHARE = 10240
FULL_CHUNKS = SHARE // CHUNK
TAIL_SHARE = E - (NW - 1) * SHARE
TAIL_CHUNKS = TAIL_SHARE // CHUNK
PHASE = 40
ROWS_PER_TILE = 624
TAIL_ROWS = N - NS * ROWS_PER_TILE


def _sc_aggregate(x, edge_index):
    mesh = plsc.VectorSubcoreMesh(core_axis_name="c", subcore_axis_name="s",
                                  num_cores=NC, num_subcores=NS)

    @functools.partial(
        pl.kernel,
        out_type=jax.ShapeDtypeStruct((NC, N, D), jnp.float32),
        mesh=mesh,
        scratch_types=[
            pltpu.VMEM_SHARED((N, D), jnp.float32),
            pltpu.VMEM((2, PHASE * CHUNK), jnp.int32),
            pltpu.VMEM((CHUNK,), jnp.int32),
            pltpu.VMEM((2, CHUNK, D), jnp.float32),
            pltpu.SemaphoreType.DMA,
        ],
    )
    def agg_kernel(x_hbm, ei_hbm, out_hbm, acc, em, cbuf, rows_v, gsem):
        c = lax.axis_index("c")
        s = lax.axis_index("s")
        w = c * NS + s
        base = pl.multiple_of(w * SHARE, CHUNK)
        nchunks = jnp.where(w == NW - 1, TAIL_CHUNKS, FULL_CHUNKS)

        r0 = s * ROWS_PER_TILE
        pltpu.sync_copy(x_hbm.at[pl.ds(r0, ROWS_PER_TILE)],
                        acc.at[pl.ds(r0, ROWS_PER_TILE)])

        @pl.when(s == 0)
        def _():
            pltpu.sync_copy(x_hbm.at[pl.ds(NS * ROWS_PER_TILE, TAIL_ROWS)],
                            acc.at[pl.ds(NS * ROWS_PER_TILE, TAIL_ROWS)])

        @pl.when(w < NW - 1)
        def _():
            pltpu.sync_copy(ei_hbm.at[:, pl.ds(base, PHASE * CHUNK)], em)

        @pl.when(w == NW - 1)
        def _():
            pltpu.sync_copy(ei_hbm.at[:, pl.ds(base, TAIL_SHARE)],
                            em.at[:, pl.ds(0, TAIL_SHARE)])

        plsc.subcore_barrier()

        def src_slice(j):
            return em.at[0, pl.ds(pl.multiple_of(lax.rem(j, PHASE) * CHUNK, 8),
                                  CHUNK)]

        pltpu.async_copy(x_hbm.at[src_slice(0)], rows_v.at[0], gsem)

        def body(j, _):
            jp = lax.rem(j, PHASE)
            for k in range(CHUNK // 16):
                cbuf[pl.ds(k * 16, 16)] = em[1, pl.ds(jp * CHUNK + k * 16, 16)]

            @pl.when(j + 1 == PHASE)
            def _():
                pltpu.make_async_copy(x_hbm.at[src_slice(j)],
                                      rows_v.at[lax.rem(j, 2)], gsem).wait()
                pltpu.sync_copy(
                    ei_hbm.at[:, pl.ds(base + PHASE * CHUNK, PHASE * CHUNK)],
                    em)
                pltpu.async_copy(x_hbm.at[src_slice(j + 1)],
                                 rows_v.at[lax.rem(j + 1, 2)], gsem)
                pltpu.sync_copy(rows_v.at[lax.rem(j, 2)], acc.at[cbuf],
                                add=True)

            @pl.when(j + 1 != PHASE)
            def _():
                @pl.when(j + 1 < nchunks)
                def _():
                    pltpu.async_copy(x_hbm.at[src_slice(j + 1)],
                                     rows_v.at[lax.rem(j + 1, 2)], gsem)

                pltpu.make_async_copy(x_hbm.at[src_slice(j)],
                                      rows_v.at[lax.rem(j, 2)], gsem).wait()
                pltpu.sync_copy(rows_v.at[lax.rem(j, 2)], acc.at[cbuf],
                                add=True)
            return 0

        lax.fori_loop(0, nchunks, body, 0)
        plsc.subcore_barrier()

        pltpu.sync_copy(acc.at[pl.ds(r0, ROWS_PER_TILE)],
                        out_hbm.at[c, pl.ds(r0, ROWS_PER_TILE)])

        @pl.when(s == 0)
        def _():
            pltpu.sync_copy(acc.at[pl.ds(NS * ROWS_PER_TILE, TAIL_ROWS)],
                            out_hbm.at[c, pl.ds(NS * ROWS_PER_TILE, TAIL_ROWS)])

    return agg_kernel(x, edge_index)


def _mlp_body(parts_ref, x_ref, w1_ref, b1_ref, gamma_ref, beta_ref, w2_ref,
              b2_ref, out_ref):
    h = parts_ref[0] + parts_ref[1] - x_ref[...]
    h1 = jnp.dot(h, w1_ref[...], preferred_element_type=jnp.float32)
    h1 = h1 + b1_ref[...]
    mean = jnp.mean(h1, axis=0, keepdims=True)
    cent = h1 - mean
    var = jnp.mean(cent * cent, axis=0, keepdims=True)
    hn = gamma_ref[...] * cent * lax.rsqrt(var + BN_EPS) + beta_ref[...]
    hr = jnp.maximum(hn, 0.0)
    out = jnp.dot(hr, w2_ref[...], preferred_element_type=jnp.float32)
    out_ref[...] = out + b2_ref[...]


def kernel(x, edge_index, edge_attr, w1, b1, gamma, beta, w2, b2):
    del edge_attr
    parts = _sc_aggregate(x, edge_index)

    out = pl.pallas_call(
        _mlp_body,
        out_shape=jax.ShapeDtypeStruct((N, D), jnp.float32),
    )(parts, x, w1, b1.reshape(1, D), gamma.reshape(1, D), beta.reshape(1, D),
      w2, b2.reshape(1, D))
    return out
